# baseline (speedup 1.0000x reference)
import functools

import jax
import jax.numpy as jnp
from jax import lax
from jax.experimental import pallas as pl
from jax.experimental.pallas import tpu as pltpu
from jax.experimental.pallas import tpu_sc as plsc


_NC = 2
_NS = 16
_NW = _NC * _NS
_L = 16


def _gate_block(x_ref, wt_ref, lt_ref, tok_ref, *, rows_per_w):
    x = x_ref[...]
    tok_ref[...] = x
    wt = wt_ref[...]
    logits = jnp.dot(x, wt, preferred_element_type=jnp.float32)
    lt = logits.T
    w_per_block = lt.shape[1] // rows_per_w
    for j in range(w_per_block):
        lt_ref[j] = lt[:, j * rows_per_w:(j + 1) * rows_per_w]


def _make_sc_top2(T, E):
    rows_per_w = T // _NW
    n_groups = rows_per_w // _L
    mesh = plsc.VectorSubcoreMesh(core_axis_name="c", subcore_axis_name="s")

    @functools.partial(
        pl.kernel,
        mesh=mesh,
        out_type=[
            jax.ShapeDtypeStruct((2 * T,), jnp.float32),
            jax.ShapeDtypeStruct((2 * T,), jnp.int32),
        ],
        scratch_types=[
            pltpu.VMEM((E * rows_per_w,), jnp.float32),
            pltpu.VMEM((rows_per_w,), jnp.float32),
            pltpu.VMEM((rows_per_w,), jnp.float32),
            pltpu.VMEM((rows_per_w,), jnp.int32),
            pltpu.VMEM((rows_per_w,), jnp.int32),
        ],
    )
    def sc_top2(logits_hbm, w_hbm, i_hbm, log_v, w1_v, w2_v, i1_v, i2_v):
        wid = lax.axis_index("s") * _NC + lax.axis_index("c")
        base = wid * rows_per_w
        pltpu.sync_copy(
            logits_hbm.at[pl.ds(base * E, rows_per_w * E)], log_v)

        neg_huge = jnp.full((_L,), -3.0e38, jnp.float32)
        zeros16 = jnp.zeros((_L,), jnp.int32)

        def group_body(g, carry):
            r0 = g * _L
            m1 = neg_huge
            m2 = neg_huge
            i1 = zeros16
            i2 = zeros16
            for e in range(E):
                v = log_v[pl.ds(e * rows_per_w + r0, _L)]
                e_vec = jnp.full((_L,), e, jnp.int32)
                gt1 = v > m1
                gt2 = v > m2
                m2 = jnp.where(gt1, m1, jnp.where(gt2, v, m2))
                i2 = jnp.where(gt1, i1, jnp.where(gt2, e_vec, i2))
                m1 = jnp.where(gt1, v, m1)
                i1 = jnp.where(gt1, e_vec, i1)
            ex = jnp.exp(m2 - m1)
            w1 = 1.0 / (1.0 + ex)
            w1_v[pl.ds(r0, _L)] = w1
            w2_v[pl.ds(r0, _L)] = 1.0 - w1
            i1_v[pl.ds(r0, _L)] = i1
            i2_v[pl.ds(r0, _L)] = i2
            return carry

        lax.fori_loop(0, n_groups, group_body, 0)
        pltpu.sync_copy(w1_v, w_hbm.at[pl.ds(base, rows_per_w)])
        pltpu.sync_copy(w2_v, w_hbm.at[pl.ds(T + base, rows_per_w)])
        pltpu.sync_copy(i1_v, i_hbm.at[pl.ds(base, rows_per_w)])
        pltpu.sync_copy(i2_v, i_hbm.at[pl.ds(T + base, rows_per_w)])

    return sc_top2


@functools.partial(jax.jit, static_argnames=("block",))
def _route(hidden_states, W, block=1024):
    H = hidden_states.shape[-1]
    tokens = hidden_states.reshape(-1, H)
    wt = W.T
    T, _ = tokens.shape
    E = wt.shape[1]
    rows_per_w = T // _NW
    w_per_block = block // rows_per_w
    grid = (T // block,)
    lt3, tok = pl.pallas_call(
        functools.partial(_gate_block, rows_per_w=rows_per_w),
        grid=grid,
        in_specs=[
            pl.BlockSpec((block, H), lambda i: (i, 0)),
            pl.BlockSpec((H, E), lambda i: (0, 0)),
        ],
        out_specs=[
            pl.BlockSpec((w_per_block, E, rows_per_w), lambda i: (i, 0, 0)),
            pl.BlockSpec((block, H), lambda i: (i, 0)),
        ],
        out_shape=[
            jax.ShapeDtypeStruct((_NW, E, rows_per_w), jnp.float32),
            jax.ShapeDtypeStruct((T, H), jnp.float32),
        ],
    )(tokens, wt)
    w_flat, i_flat = _make_sc_top2(T, E)(lt3.reshape(-1))
    w_out = w_flat.reshape(2, T).T
    i_out = i_flat.reshape(2, T).T
    return w_out, i_out, tok


def kernel(hidden_states, W):
    w_out, i_out, tokens = _route(hidden_states, W)
    return (w_out, i_out, tokens)


# speedup vs baseline: 1.1375x; 1.1375x over previous
import functools

import jax
import jax.numpy as jnp
from jax.experimental import pallas as pl


def _gate_block(x_ref, wt_ref, w_out_ref, i_out_ref, tok_ref):
    x = x_ref[...]
    tok_ref[...] = x
    wt = wt_ref[...]
    logits = jnp.dot(x, wt, preferred_element_type=jnp.float32)
    B, E = logits.shape
    iota = jax.lax.broadcasted_iota(jnp.int32, (B, E), 1)
    big = jnp.int32(E)

    m1 = jnp.max(logits, axis=-1, keepdims=True)
    i1 = jnp.min(jnp.where(logits == m1, iota, big), axis=-1, keepdims=True)
    masked = jnp.where(iota == i1, -jnp.inf, logits)
    m2 = jnp.max(masked, axis=-1, keepdims=True)
    i2 = jnp.min(jnp.where(masked == m2, iota, big), axis=-1, keepdims=True)

    e = jnp.exp(m2 - m1)
    w1 = 1.0 / (1.0 + e)
    w2 = 1.0 - w1
    w_out_ref[...] = jnp.concatenate([w1, w2], axis=-1)
    i_out_ref[...] = jnp.concatenate([i1, i2], axis=-1)


@functools.partial(jax.jit, static_argnames=("block",))
def _route(hidden_states, W, block=1024):
    H = hidden_states.shape[-1]
    tokens = hidden_states.reshape(-1, H)
    wt = W.T
    T, _ = tokens.shape
    E = wt.shape[1]
    grid = (T // block,)
    return pl.pallas_call(
        _gate_block,
        grid=grid,
        in_specs=[
            pl.BlockSpec((block, H), lambda i: (i, 0)),
            pl.BlockSpec((H, E), lambda i: (0, 0)),
        ],
        out_specs=[
            pl.BlockSpec((block, 2), lambda i: (i, 0)),
            pl.BlockSpec((block, 2), lambda i: (i, 0)),
            pl.BlockSpec((block, H), lambda i: (i, 0)),
        ],
        out_shape=[
            jax.ShapeDtypeStruct((T, 2), jnp.float32),
            jax.ShapeDtypeStruct((T, 2), jnp.int32),
            jax.ShapeDtypeStruct((T, H), jnp.float32),
        ],
    )(tokens, wt)


def kernel(hidden_states, W):
    w_out, i_out, tokens = _route(hidden_states, W)
    return (w_out, i_out, tokens)
